# baseline (speedup 1.0000x reference)
import dataclasses

import jax
import jax.numpy as jnp
from jax import lax
from jax.experimental import pallas as pl
from jax.experimental.pallas import tpu as pltpu
from jax.experimental.pallas import tpu_sc as plsc

ROWS = 8192
DIM = 4096

NC = 2
NS = 16
NW = NC * NS
R_PER_W = ROWS // NW
CH = 8
QTR = CH // 4
NCH = R_PER_W // CH
NGRP = DIM // 16
UNROLL = 8


def _shuffle2(perm_v, in_b, out_b, lr0):
    rvecs = [jnp.full((16,), lr0 + rr, jnp.int32) for rr in range(QTR)]

    @pl.loop(0, NGRP, step=UNROLL)
    def _(j):
        base = j * 16
        idxs = [perm_v[pl.ds(base + u * 16, 16)] for u in range(UNROLL)]
        vals = [
            plsc.load_gather(in_b, [rvecs[rr], idxs[u]])
            for rr in range(QTR)
            for u in range(UNROLL)
        ]
        k = 0
        for rr in range(QTR):
            for u in range(UNROLL):
                out_b[rr, pl.ds(base + u * 16, 16)] = vals[k]
                k += 1


def _sc_body(x_hbm, perm_hbm, o_hbm, perm_v, in0, in1,
             oq0, oq1, oq2, oq3, si0, si1, sq0, sq1, sq2, sq3):
    outs = [oq0, oq1, oq2, oq3]
    sqs = [sq0, sq1, sq2, sq3]

    wid = lax.axis_index("s") * NC + lax.axis_index("c")
    base = wid * R_PER_W

    pltpu.sync_copy(perm_hbm, perm_v)

    def chunk(c):
        return pl.ds(base + c * CH, CH)

    def quarter(c, q):
        return pl.ds(base + c * CH + q * QTR, QTR)

    pltpu.async_copy(x_hbm.at[chunk(0)], in0, si0)

    @pl.loop(0, NCH, step=2)
    def _(c):
        pltpu.async_copy(x_hbm.at[chunk(c + 1)], in1, si1)
        pltpu.make_async_copy(x_hbm.at[chunk(c)], in0, si0).wait()

        for q in range(4):
            @pl.when(c >= 1)
            def _(q=q):
                pltpu.make_async_copy(
                    outs[q], o_hbm.at[quarter(c - 1, q)], sqs[q]
                ).wait()

            _shuffle2(perm_v, in0, outs[q], q * QTR)
            pltpu.async_copy(outs[q], o_hbm.at[quarter(c, q)], sqs[q])

        @pl.when(c + 2 < NCH)
        def _():
            pltpu.async_copy(x_hbm.at[chunk(c + 2)], in0, si0)

        pltpu.make_async_copy(x_hbm.at[chunk(c + 1)], in1, si1).wait()

        for q in range(4):
            pltpu.make_async_copy(
                outs[q], o_hbm.at[quarter(c, q)], sqs[q]
            ).wait()
            _shuffle2(perm_v, in1, outs[q], q * QTR)
            pltpu.async_copy(outs[q], o_hbm.at[quarter(c + 1, q)], sqs[q])

    for q in range(4):
        pltpu.make_async_copy(
            outs[q], o_hbm.at[quarter(NCH - 1, q)], sqs[q]
        ).wait()


def kernel(x, perm):
    mesh = plsc.VectorSubcoreMesh(core_axis_name="c", subcore_axis_name="s")
    cp = pltpu.CompilerParams()
    if "needs_layout_passes" in pltpu.CompilerParams.__dataclass_fields__:
        cp = dataclasses.replace(cp, needs_layout_passes=False)
    kfn = pl.kernel(
        _sc_body,
        mesh=mesh,
        compiler_params=cp,
        out_type=jax.ShapeDtypeStruct((ROWS, DIM), jnp.float32),
        scratch_types=[
            pltpu.VMEM((DIM,), jnp.int32),
            pltpu.VMEM((CH, DIM), jnp.float32),
            pltpu.VMEM((CH, DIM), jnp.float32),
            pltpu.VMEM((QTR, DIM), jnp.float32),
            pltpu.VMEM((QTR, DIM), jnp.float32),
            pltpu.VMEM((QTR, DIM), jnp.float32),
            pltpu.VMEM((QTR, DIM), jnp.float32),
            pltpu.SemaphoreType.DMA,
            pltpu.SemaphoreType.DMA,
            pltpu.SemaphoreType.DMA,
            pltpu.SemaphoreType.DMA,
            pltpu.SemaphoreType.DMA,
            pltpu.SemaphoreType.DMA,
        ],
    )
    return kfn(x, perm)


# speedup vs baseline: 1.1697x; 1.1697x over previous
import dataclasses

import jax
import jax.numpy as jnp
from jax import lax
from jax.experimental import pallas as pl
from jax.experimental.pallas import tpu as pltpu
from jax.experimental.pallas import tpu_sc as plsc

ROWS = 8192
DIM = 4096

NC = 2
NS = 16
NW = NC * NS
R_PER_W = ROWS // NW
CH = 8
HALF = CH // 2
NCH = R_PER_W // CH
NGRP = DIM // 16
UNROLL = 8


def _shuffle4(perm_v, in_b, out_b, lr0):
    rvecs = [jnp.full((16,), lr0 + rr, jnp.int32) for rr in range(HALF)]

    @pl.loop(0, NGRP, step=UNROLL)
    def _(j):
        base = j * 16
        idxs = [perm_v[pl.ds(base + u * 16, 16)] for u in range(UNROLL)]
        vals = [
            plsc.load_gather(in_b, [rvecs[rr], idxs[u]])
            for rr in range(HALF)
            for u in range(UNROLL)
        ]
        k = 0
        for rr in range(HALF):
            for u in range(UNROLL):
                out_b[rr, pl.ds(base + u * 16, 16)] = vals[k]
                k += 1


def _sc_body(x_hbm, perm_hbm, o_hbm, perm_v, in0, in1, outa, outb,
             si0, si1, soa, sob):
    wid = lax.axis_index("s") * NC + lax.axis_index("c")
    base = wid * R_PER_W

    pltpu.sync_copy(perm_hbm, perm_v)

    def chunk(c):
        return pl.ds(base + c * CH, CH)

    def half_a(c):
        return pl.ds(base + c * CH, HALF)

    def half_b(c):
        return pl.ds(base + c * CH + HALF, HALF)

    pltpu.async_copy(x_hbm.at[chunk(0)], in0, si0)

    @pl.loop(0, NCH, step=2)
    def _(c):
        pltpu.async_copy(x_hbm.at[chunk(c + 1)], in1, si1)
        pltpu.make_async_copy(x_hbm.at[chunk(c)], in0, si0).wait()

        @pl.when(c >= 1)
        def _():
            pltpu.make_async_copy(outa, o_hbm.at[half_a(c - 1)], soa).wait()

        _shuffle4(perm_v, in0, outa, 0)
        pltpu.async_copy(outa, o_hbm.at[half_a(c)], soa)

        @pl.when(c >= 1)
        def _():
            pltpu.make_async_copy(outb, o_hbm.at[half_b(c - 1)], sob).wait()

        _shuffle4(perm_v, in0, outb, HALF)
        pltpu.async_copy(outb, o_hbm.at[half_b(c)], sob)

        @pl.when(c + 2 < NCH)
        def _():
            pltpu.async_copy(x_hbm.at[chunk(c + 2)], in0, si0)

        pltpu.make_async_copy(x_hbm.at[chunk(c + 1)], in1, si1).wait()

        pltpu.make_async_copy(outa, o_hbm.at[half_a(c)], soa).wait()
        _shuffle4(perm_v, in1, outa, 0)
        pltpu.async_copy(outa, o_hbm.at[half_a(c + 1)], soa)

        pltpu.make_async_copy(outb, o_hbm.at[half_b(c)], sob).wait()
        _shuffle4(perm_v, in1, outb, HALF)
        pltpu.async_copy(outb, o_hbm.at[half_b(c + 1)], sob)

    pltpu.make_async_copy(outa, o_hbm.at[half_a(NCH - 1)], soa).wait()
    pltpu.make_async_copy(outb, o_hbm.at[half_b(NCH - 1)], sob).wait()


def kernel(x, perm):
    mesh = plsc.VectorSubcoreMesh(core_axis_name="c", subcore_axis_name="s")
    cp = pltpu.CompilerParams()
    if "needs_layout_passes" in pltpu.CompilerParams.__dataclass_fields__:
        cp = dataclasses.replace(cp, needs_layout_passes=False)
    kfn = pl.kernel(
        _sc_body,
        mesh=mesh,
        compiler_params=cp,
        out_type=jax.ShapeDtypeStruct((ROWS, DIM), jnp.float32),
        scratch_types=[
            pltpu.VMEM((DIM,), jnp.int32),
            pltpu.VMEM((CH, DIM), jnp.float32),
            pltpu.VMEM((CH, DIM), jnp.float32),
            pltpu.VMEM((HALF, DIM), jnp.float32),
            pltpu.VMEM((HALF, DIM), jnp.float32),
            pltpu.SemaphoreType.DMA,
            pltpu.SemaphoreType.DMA,
            pltpu.SemaphoreType.DMA,
            pltpu.SemaphoreType.DMA,
        ],
    )
    return kfn(x, perm)
